# baseline (speedup 1.0000x reference)
import functools

import jax
import jax.numpy as jnp
from jax import lax
from jax.experimental import pallas as pl
from jax.experimental.pallas import tpu as pltpu
from jax.experimental.pallas import tpu_sc as plsc

_NC = 2
_NS = 16
_NW = _NC * _NS
_CH = 32


def _sc_body(x_hbm, pos_hbm, table_hbm, out_hbm, idx_v, tbuf, buf, sem):
    B, S, D = x_hbm.shape
    pos_w = S // _NW
    nch = pos_w // _CH
    nsl = D // 16
    wid = lax.axis_index("s") * _NC + lax.axis_index("c")

    def chunk(c, carry):
        seq = wid * pos_w + c * _CH
        pltpu.sync_copy(pos_hbm.at[pl.ds(seq, _CH)], idx_v)
        pltpu.async_copy(table_hbm.at[idx_v], tbuf, sem).wait()
        for b in range(B):
            pltpu.sync_copy(x_hbm.at[b, pl.ds(seq, _CH)], buf)

            def row(r, carry2):
                for s in range(nsl):
                    sl = pl.ds(s * 16, 16)
                    buf[r, sl] = buf[r, sl] + tbuf[r, sl]
                return carry2

            lax.fori_loop(0, _CH, row, 0)
            pltpu.sync_copy(buf, out_hbm.at[b, pl.ds(seq, _CH)])
        return carry

    lax.fori_loop(0, nch, chunk, 0)


@jax.jit
def _posemb_add_sc(x, pos, table):
    B, S, D = x.shape
    k = pl.kernel(
        _sc_body,
        out_type=jax.ShapeDtypeStruct((B, S, D), x.dtype),
        mesh=plsc.VectorSubcoreMesh(core_axis_name="c", subcore_axis_name="s"),
        scratch_types=[
            pltpu.VMEM((_CH,), jnp.int32),
            pltpu.VMEM((_CH, D), jnp.float32),
            pltpu.VMEM((_CH, D), jnp.float32),
            pltpu.SemaphoreType.DMA,
        ],
    )
    return k(x, pos, table)


def kernel(x, table, offset=0):
    B, S, D = x.shape
    pos = offset + jnp.arange(S, dtype=jnp.int32)
    return _posemb_add_sc(x, pos, table)


# speedup vs baseline: 1.2375x; 1.2375x over previous
import functools

import jax
import jax.numpy as jnp
from jax import lax
from jax.experimental import pallas as pl
from jax.experimental.pallas import tpu as pltpu
from jax.experimental.pallas import tpu_sc as plsc

_NC = 2
_NS = 16
_NW = _NC * _NS
_CH = 32
_BS = 2048


def _sc_body(x_hbm, pos_hbm, table_hbm, out_hbm, idx_v, tbuf, buf, sem):
    _, S, D = x_hbm.shape
    pos_w = S // _NW
    nch = pos_w // _CH
    nsl = D // 16
    wid = lax.axis_index("s") * _NC + lax.axis_index("c")

    def chunk(c, carry):
        seq = wid * pos_w + c * _CH
        pltpu.sync_copy(pos_hbm.at[pl.ds(seq, _CH)], idx_v)
        pltpu.async_copy(table_hbm.at[idx_v], tbuf, sem).wait()
        pltpu.sync_copy(x_hbm.at[0, pl.ds(seq, _CH)], buf)

        def row(r, carry2):
            for s in range(nsl):
                sl = pl.ds(s * 16, 16)
                buf[r, sl] = buf[r, sl] + tbuf[r, sl]
            return carry2

        lax.fori_loop(0, _CH, row, 0)
        pltpu.sync_copy(buf, out_hbm.at[0, pl.ds(seq, _CH)])
        return carry

    lax.fori_loop(0, nch, chunk, 0)


def _add_block(x_ref, t_ref, o_ref):
    o_ref[...] = x_ref[...] + t_ref[...][None]


@jax.jit
def _posemb_add_hybrid(x, pos, table, table_slice):
    B, S, D = x.shape
    sc = pl.kernel(
        _sc_body,
        out_type=jax.ShapeDtypeStruct((1, S, D), x.dtype),
        mesh=plsc.VectorSubcoreMesh(core_axis_name="c", subcore_axis_name="s"),
        scratch_types=[
            pltpu.VMEM((_CH,), jnp.int32),
            pltpu.VMEM((_CH, D), jnp.float32),
            pltpu.VMEM((_CH, D), jnp.float32),
            pltpu.SemaphoreType.DMA,
        ],
    )
    sc_out = sc(x, pos, table)
    tc_out = pl.pallas_call(
        _add_block,
        grid=(S // _BS, B - 1),
        in_specs=[
            pl.BlockSpec((1, _BS, D), lambda i, b: (b + 1, i, 0)),
            pl.BlockSpec((_BS, D), lambda i, b: (i, 0)),
        ],
        out_specs=pl.BlockSpec((1, _BS, D), lambda i, b: (b, i, 0)),
        out_shape=jax.ShapeDtypeStruct((B - 1, S, D), x.dtype),
        compiler_params=pltpu.CompilerParams(
            dimension_semantics=("parallel", "parallel"),
        ),
    )(x, table_slice)
    return jnp.concatenate([sc_out, tc_out], axis=0)


def kernel(x, table, offset=0):
    B, S, D = x.shape
    pos = offset + jnp.arange(S, dtype=jnp.int32)
    table_slice = jax.lax.dynamic_slice_in_dim(table, offset, S, axis=0)
    return _posemb_add_hybrid(x, pos, table, table_slice)


# speedup vs baseline: 2.8203x; 2.2791x over previous
import functools

import jax
import jax.numpy as jnp
from jax.experimental import pallas as pl
from jax.experimental.pallas import tpu as pltpu

_BS = 2048


def _add_block(x_ref, t_ref, o_ref):
    o_ref[...] = x_ref[...] + t_ref[...][None]


@functools.partial(jax.jit, static_argnums=(2,))
def _posemb_add(x, table_slice, seq_block):
    B, S, D = x.shape
    n_seq = S // seq_block
    return pl.pallas_call(
        _add_block,
        grid=(n_seq, B),
        in_specs=[
            pl.BlockSpec((1, seq_block, D), lambda i, b: (b, i, 0)),
            pl.BlockSpec((seq_block, D), lambda i, b: (i, 0)),
        ],
        out_specs=pl.BlockSpec((1, seq_block, D), lambda i, b: (b, i, 0)),
        out_shape=jax.ShapeDtypeStruct((B, S, D), x.dtype),
        compiler_params=pltpu.CompilerParams(
            dimension_semantics=("parallel", "parallel"),
            vmem_limit_bytes=128 * 1024 * 1024,
        ),
    )(x, table_slice)


def kernel(x, table, offset=0):
    S = x.shape[1]
    table_slice = jax.lax.dynamic_slice_in_dim(table, offset, S, axis=0)
    return _posemb_add(x, table_slice, _BS)
